# baseline (speedup 1.0000x reference)
import functools

import jax
import jax.numpy as jnp
from jax import lax
from jax.experimental import pallas as pl
from jax.experimental.pallas import tpu as pltpu
from jax.experimental.pallas import tpu_sc as plsc

_NUM_ITEMS = 1000000
_EMBED = 32
_BLK = 131072


def _proj_body(xt_ref, w_ref, b_ref, o_ref):
    o_ref[...] = jnp.sum(xt_ref[...] * w_ref[...], axis=0) + b_ref[0]


def _project(table_t, w_col, b):
    return pl.pallas_call(
        _proj_body,
        grid=(pl.cdiv(_NUM_ITEMS, _BLK),),
        in_specs=[
            pl.BlockSpec((_EMBED, _BLK), lambda i: (0, i)),
            pl.BlockSpec((_EMBED, 1), lambda i: (0, 0)),
            pl.BlockSpec(memory_space=pltpu.SMEM),
        ],
        out_specs=pl.BlockSpec((_BLK,), lambda i: (i,)),
        out_shape=jax.ShapeDtypeStruct((_NUM_ITEMS,), jnp.float32),
    )(table_t, w_col, b)


@functools.cache
def _make_gather(num_elems):
    info = plsc.get_sparse_core_info()
    nc, ns = info.num_cores, info.num_subcores
    nw = nc * ns
    per_w = num_elems // nw
    assert per_w * nw == num_elems and per_w % 8 == 0
    mesh = plsc.VectorSubcoreMesh(core_axis_name="c", subcore_axis_name="s")

    nchunks = 4
    chunk = per_w // nchunks
    assert chunk * nchunks == per_w and chunk % 8 == 0

    @functools.partial(
        pl.kernel,
        mesh=mesh,
        out_type=jax.ShapeDtypeStruct((num_elems,), jnp.float32),
        scratch_types=[
            pltpu.VMEM((chunk,), jnp.int32),
            pltpu.VMEM((chunk,), jnp.int32),
            pltpu.VMEM((chunk,), jnp.float32),
            pltpu.VMEM((chunk,), jnp.float32),
            pltpu.SemaphoreType.DMA,
        ],
    )
    def gather_k(proj_hbm, idx_hbm, out_hbm, idx_a, idx_b, val_a, val_b, sem):
        wid = lax.axis_index("s") * nc + lax.axis_index("c")
        base = wid * per_w
        idx_bufs = (idx_a, idx_b)
        val_bufs = (val_a, val_b)
        pltpu.sync_copy(idx_hbm.at[pl.ds(base, chunk)], idx_bufs[0])
        cp = pltpu.async_copy(proj_hbm.at[idx_bufs[0]], val_bufs[0], sem)
        for c in range(1, nchunks):
            pltpu.sync_copy(
                idx_hbm.at[pl.ds(base + c * chunk, chunk)], idx_bufs[c % 2]
            )
            cp_next = pltpu.async_copy(
                proj_hbm.at[idx_bufs[c % 2]], val_bufs[c % 2], sem
            )
            cp.wait()
            pltpu.sync_copy(
                val_bufs[(c - 1) % 2],
                out_hbm.at[pl.ds(base + (c - 1) * chunk, chunk)],
            )
            cp = cp_next
        cp.wait()
        pltpu.sync_copy(
            val_bufs[(nchunks - 1) % 2],
            out_hbm.at[pl.ds(base + (nchunks - 1) * chunk, chunk)],
        )

    return gather_k


def kernel(item_ids, table, W, b):
    bsz, hist = item_ids.shape
    num_elems = bsz * hist
    proj = _project(table.T, W.reshape(_EMBED, 1), b)
    flat = _make_gather(num_elems)(
        proj, item_ids.reshape(num_elems).astype(jnp.int32)
    )
    return flat.reshape(bsz, hist, 1)


# speedup vs baseline: 1.0137x; 1.0137x over previous
import functools

import jax
import jax.numpy as jnp
from jax import lax
from jax.experimental import pallas as pl
from jax.experimental.pallas import tpu as pltpu
from jax.experimental.pallas import tpu_sc as plsc

_NUM_ITEMS = 1000000
_EMBED = 32
_BLK = 131072


def _proj_body(xt_ref, w_ref, b_ref, o_ref):
    o_ref[...] = jnp.sum(xt_ref[...] * w_ref[...], axis=0) + b_ref[0]


def _project(table_t, w_col, b):
    return pl.pallas_call(
        _proj_body,
        grid=(pl.cdiv(_NUM_ITEMS, _BLK),),
        in_specs=[
            pl.BlockSpec((_EMBED, _BLK), lambda i: (0, i)),
            pl.BlockSpec((_EMBED, 1), lambda i: (0, 0)),
            pl.BlockSpec(memory_space=pltpu.SMEM),
        ],
        out_specs=pl.BlockSpec((_BLK,), lambda i: (i,)),
        out_shape=jax.ShapeDtypeStruct((_NUM_ITEMS,), jnp.float32),
    )(table_t, w_col, b)


@functools.cache
def _make_gather(num_elems):
    info = plsc.get_sparse_core_info()
    nc, ns = info.num_cores, info.num_subcores
    nw = nc * ns
    per_w = num_elems // nw
    assert per_w * nw == num_elems and per_w % 8 == 0
    mesh = plsc.VectorSubcoreMesh(core_axis_name="c", subcore_axis_name="s")

    @functools.partial(
        pl.kernel,
        mesh=mesh,
        out_type=jax.ShapeDtypeStruct((num_elems,), jnp.float32),
        scratch_types=[
            pltpu.VMEM((per_w,), jnp.int32),
            pltpu.VMEM((per_w,), jnp.float32),
            pltpu.SemaphoreType.DMA,
        ],
    )
    def gather_k(proj_hbm, idx_hbm, out_hbm, idx_v, vals_v, sem):
        wid = lax.axis_index("s") * nc + lax.axis_index("c")
        base = wid * per_w
        pltpu.sync_copy(idx_hbm.at[pl.ds(base, per_w)], idx_v)
        pltpu.async_copy(proj_hbm.at[idx_v], vals_v, sem).wait()
        pltpu.sync_copy(vals_v, out_hbm.at[pl.ds(base, per_w)])

    return gather_k


def kernel(item_ids, table, W, b):
    bsz, hist = item_ids.shape
    num_elems = bsz * hist
    proj = _project(table.T, W.reshape(_EMBED, 1), b)
    flat = _make_gather(num_elems)(
        proj, item_ids.reshape(num_elems).astype(jnp.int32)
    )
    return flat.reshape(bsz, hist, 1)
